# baseline (speedup 1.0000x reference)
import functools
import jax
import jax.numpy as jnp
from jax import lax
from jax.experimental import pallas as pl
from jax.experimental.pallas import tpu as pltpu
from jax.experimental.pallas import tpu_sc as plsc

NC, NS, L = 2, 16, 16
NW = NC * NS

R_CH = 8
NBUF = 2


def _sc_body(n_ch, rows_w, n_b, data_hbm, pos_hbm, out_hbm,
             pos_v, bufs, in_sems, out_sems):
    wid = lax.axis_index("s") * NC + lax.axis_index("c")
    base = wid * rows_w

    pltpu.sync_copy(pos_hbm.at[pl.ds(base * L, rows_w * L)], pos_v)

    def rows_of(c):
        return pl.ds(base + c * R_CH, R_CH)

    def fire_in(b, c):
        pltpu.async_copy(data_hbm.at[rows_of(c)], bufs.at[b], in_sems.at[b])

    def wait_in(b, c):
        pltpu.make_async_copy(
            data_hbm.at[rows_of(c)], bufs.at[b], in_sems.at[b]).wait()

    def fire_out(b, c):
        pltpu.async_copy(bufs.at[b], out_hbm.at[rows_of(c)], out_sems.at[b])

    def wait_out(b, c):
        pltpu.make_async_copy(
            bufs.at[b], out_hbm.at[rows_of(c)], out_sems.at[b]).wait()

    def compute(b, c):
        row0 = c * R_CH
        splats = []
        for r in range(R_CH):
            splats.append(pos_v[pl.ds((row0 + r) * L, L)])

        def step(t, carry):
            base = t * 128
            for r in range(R_CH):
                for g in range(128 // L):
                    sl = pl.ds(base + g * L, L)
                    bufs[b, r, sl] = bufs[b, r, sl] + splats[r]
            return carry

        lax.fori_loop(0, n_b // 128, step, 0)

    for b in range(NBUF - 1):
        fire_in(b, b)

    for b in range(NBUF):
        wait_in(b, b)
        compute(b, b)
        fire_out(b, b)
        bprev = (b + NBUF - 1) % NBUF
        if b == 0:
            fire_in(NBUF - 1, NBUF - 1)
        else:
            wait_out(bprev, b - 1)
            fire_in(bprev, b - 1 + NBUF)

    def grp(g, carry):
        for b in range(NBUF):
            c = g * NBUF + b
            wait_in(b, c)
            compute(b, c)
            fire_out(b, c)
            bprev = (b + NBUF - 1) % NBUF
            wait_out(bprev, c - 1)
            fire_in(bprev, jnp.minimum(c - 1 + NBUF, n_ch - 1))
        return carry

    lax.fori_loop(1, n_ch // NBUF, grp, 0)

    for b in range(NBUF - 1):
        wait_in(b, n_ch - 1)
    wait_out(NBUF - 1, n_ch - 1)


def kernel(data, pos_emb_weight):
    B, S, E = data.shape
    R = S * E
    dt = jnp.transpose(data, (1, 2, 0)).reshape(R, B)
    pos_splat = jnp.broadcast_to(
        pos_emb_weight[:S].reshape(R)[:, None], (R, L)).reshape(R * L)
    rows_w = R // NW
    n_ch = rows_w // R_CH

    mesh = plsc.VectorSubcoreMesh(
        core_axis_name="c", subcore_axis_name="s",
        num_cores=NC, num_subcores=NS)
    body = functools.partial(_sc_body, n_ch, rows_w, B)
    out_t = pl.kernel(
        body,
        out_type=jax.ShapeDtypeStruct((R, B), jnp.float32),
        mesh=mesh,
        scratch_types=[
            pltpu.VMEM((rows_w * L,), jnp.float32),
            pltpu.VMEM((NBUF, R_CH, B), jnp.float32),
            pltpu.SemaphoreType.DMA((NBUF,)),
            pltpu.SemaphoreType.DMA((NBUF,)),
        ],
    )(dt, pos_splat)
    return jnp.transpose(out_t.reshape(S, E, B), (2, 0, 1))


# speedup vs baseline: 1.2316x; 1.2316x over previous
import functools
import jax
import jax.numpy as jnp
from jax import lax
from jax.experimental import pallas as pl
from jax.experimental.pallas import tpu as pltpu
from jax.experimental.pallas import tpu_sc as plsc

NC, NS, L = 2, 16, 16
NW = NC * NS

R_CH = 8
NBUF = 3


def _sc_body(n_ch, rows_w, n_b, data_hbm, pos_hbm, out_hbm,
             pos_v, bufs, in_sems, out_sems):
    wid = lax.axis_index("s") * NC + lax.axis_index("c")
    base = wid * rows_w

    pltpu.sync_copy(pos_hbm.at[pl.ds(base * L, rows_w * L)], pos_v)

    def rows_of(c):
        return pl.ds(base + c * R_CH, R_CH)

    def fire_in(b, c):
        pltpu.async_copy(data_hbm.at[rows_of(c)], bufs.at[b], in_sems.at[b])

    def wait_in(b, c):
        pltpu.make_async_copy(
            data_hbm.at[rows_of(c)], bufs.at[b], in_sems.at[b]).wait()

    def fire_out(b, c):
        pltpu.async_copy(bufs.at[b], out_hbm.at[rows_of(c)], out_sems.at[b])

    def wait_out(b, c):
        pltpu.make_async_copy(
            bufs.at[b], out_hbm.at[rows_of(c)], out_sems.at[b]).wait()

    def compute(b, c):
        row0 = c * R_CH
        splats = []
        for r in range(R_CH):
            splats.append(pos_v[pl.ds((row0 + r) * L, L)])

        def step(t, carry):
            base = t * 128
            for r in range(R_CH):
                for g in range(128 // L):
                    sl = pl.ds(base + g * L, L)
                    bufs[b, r, sl] = bufs[b, r, sl] + splats[r]
            return carry

        lax.fori_loop(0, n_b // 128, step, 0)

    for b in range(NBUF - 1):
        fire_in(b, b)

    for b in range(NBUF):
        wait_in(b, b)
        compute(b, b)
        fire_out(b, b)
        bprev = (b + NBUF - 1) % NBUF
        if b == 0:
            fire_in(NBUF - 1, NBUF - 1)
        else:
            wait_out(bprev, b - 1)
            fire_in(bprev, b - 1 + NBUF)

    def grp(g, carry):
        for b in range(NBUF):
            c = g * NBUF + b
            wait_in(b, c)
            compute(b, c)
            fire_out(b, c)
            bprev = (b + NBUF - 1) % NBUF
            wait_out(bprev, c - 1)
            fire_in(bprev, jnp.minimum(c - 1 + NBUF, n_ch - 1))
        return carry

    n_grp = n_ch // NBUF
    lax.fori_loop(1, n_grp, grp, 0)

    for c in range(n_grp * NBUF, n_ch):
        b = c % NBUF
        wait_in(b, c)
        compute(b, c)
        fire_out(b, c)
        bprev = (b + NBUF - 1) % NBUF
        wait_out(bprev, c - 1)
        fire_in(bprev, jnp.minimum(c - 1 + NBUF, n_ch - 1))

    b_last = (n_ch - 1) % NBUF
    for b in range(NBUF):
        if b != b_last:
            wait_in(b, n_ch - 1)
    wait_out(b_last, n_ch - 1)


def kernel(data, pos_emb_weight):
    B, S, E = data.shape
    R = S * E
    dt = jnp.transpose(data, (1, 2, 0)).reshape(R, B)
    pos_splat = jnp.broadcast_to(
        pos_emb_weight[:S].reshape(R)[:, None], (R, L)).reshape(R * L)
    rows_w = R // NW
    n_ch = rows_w // R_CH

    mesh = plsc.VectorSubcoreMesh(
        core_axis_name="c", subcore_axis_name="s",
        num_cores=NC, num_subcores=NS)
    body = functools.partial(_sc_body, n_ch, rows_w, B)
    out_t = pl.kernel(
        body,
        out_type=jax.ShapeDtypeStruct((R, B), jnp.float32),
        mesh=mesh,
        scratch_types=[
            pltpu.VMEM((rows_w * L,), jnp.float32),
            pltpu.VMEM((NBUF, R_CH, B), jnp.float32),
            pltpu.SemaphoreType.DMA((NBUF,)),
            pltpu.SemaphoreType.DMA((NBUF,)),
        ],
    )(dt, pos_splat)
    return jnp.transpose(out_t.reshape(S, E, B), (2, 0, 1))
